# Initial kernel scaffold; baseline (speedup 1.0000x reference)
import jax
import jax.numpy as jnp
from jax.experimental import pallas as pl


def kernel(x, edge_index, batch, W0, b0, W1, b1, W2, b2, W3, b3, Wout, bout):
    raise NotImplementedError("write your pallas kernel here")


# speedup vs baseline: 2.2793x; 2.2793x over previous
import jax
import jax.numpy as jnp
from jax.experimental import pallas as pl

_N = 10000
_BLK = 2000


def _mm_body(a_ref, w_ref, o_ref):
    o_ref[...] = jnp.dot(a_ref[...], w_ref[...], preferred_element_type=jnp.float32)


def _mm(a, w):
    n, d = a.shape
    h = w.shape[1]
    return pl.pallas_call(
        _mm_body,
        grid=(n // _BLK,),
        in_specs=[
            pl.BlockSpec((_BLK, d), lambda i: (i, 0)),
            pl.BlockSpec((d, h), lambda i: (0, 0)),
        ],
        out_specs=pl.BlockSpec((_BLK, h), lambda i: (i, 0)),
        out_shape=jax.ShapeDtypeStruct((n, h), jnp.float32),
    )(a, w)


def _gcn(x, s, d, dinv, W, b):
    u = _mm(x, W)
    z = dinv[:, None] * u
    agg = jnp.zeros_like(z).at[d].add(z[s])
    return dinv[:, None] * (agg + z) + b


def kernel(x, edge_index, batch, W0, b0, W1, b1, W2, b2, W3, b3, Wout, bout):
    s = edge_index[0]
    d = edge_index[1]
    deg = jnp.zeros((_N,), jnp.float32).at[d].add(1.0) + 1.0
    dinv = jax.lax.rsqrt(deg)
    h = jax.nn.relu(_gcn(x, s, d, dinv, W0, b0))
    h = jax.nn.relu(_gcn(h, s, d, dinv, W1, b1))
    h = jax.nn.relu(_gcn(h, s, d, dinv, W2, b2))
    h = jax.nn.relu(_gcn(h, s, d, dinv, W3, b3))
    sums = jax.ops.segment_sum(h, batch, num_segments=64)
    cnt = jax.ops.segment_sum(jnp.ones((_N, 1), h.dtype), batch, num_segments=64)
    pooled = sums / jnp.maximum(cnt, 1.0)
    return _mm_small(pooled, Wout, bout)


def _out_body(p_ref, w_ref, b_ref, o_ref):
    o_ref[...] = jnp.dot(p_ref[...], w_ref[...], preferred_element_type=jnp.float32) + b_ref[...]


def _mm_small(p, w, b):
    g, h = p.shape
    t = w.shape[1]
    return pl.pallas_call(
        _out_body,
        out_shape=jax.ShapeDtypeStruct((g, t), jnp.float32),
    )(p, w, b[None, :])
